# Initial kernel scaffold; baseline (speedup 1.0000x reference)
import functools

import jax
import jax.numpy as jnp
from jax import lax
from jax.experimental import pallas as pl
from jax.experimental.pallas import tpu as pltpu
from jax.experimental.pallas import tpu_sc as plsc

LANE = 128
NSC = 2
NTILE = 16
NW = NSC * NTILE
NB = 80
EP = NW * NB * LANE
NP = 10240
PER_T = NP // NTILE
ZR = 128
BR = 1000


def _deg_body(dstp_hbm, outdeg_hbm, didx, ones_v, zvec, deg_sh):
    cid = lax.axis_index("c")
    sid = lax.axis_index("s")
    wid = cid * NTILE + sid
    zero = jnp.zeros((16,), jnp.float32)
    one = jnp.ones((16,), jnp.float32)
    for i in range(PER_T // 16):
        zvec[pl.ds(i * 16, 16)] = zero
    for i in range(LANE // 16):
        ones_v[pl.ds(i * 16, 16)] = one
    pltpu.sync_copy(zvec, deg_sh.at[pl.ds(sid * PER_T, PER_T)])
    pltpu.sync_copy(dstp_hbm.at[pl.ds(wid * NB, NB)], didx)
    plsc.subcore_barrier()

    def body(j, c):
        pltpu.sync_copy(ones_v, deg_sh.at[didx.at[j]], add=True)
        return c

    lax.fori_loop(0, NB, body, 0)
    plsc.subcore_barrier()

    @pl.when(sid == 0)
    def _():
        pltpu.sync_copy(deg_sh, outdeg_hbm.at[cid])


def _scat_body(g_hbm, srcp_hbm, dstp_hbm, out_hbm,
               sidx, didx, bufa, bufb, zbuf, s_sh, sem_a, sem_b):
    cid = lax.axis_index("c")
    sid = lax.axis_index("s")
    wid = cid * NTILE + sid
    zero = jnp.zeros((16,), jnp.float32)

    def zrow(i, c):
        for k in range(LANE // 16):
            zbuf[i, pl.ds(k * 16, 16)] = zero
        return c

    lax.fori_loop(0, ZR, zrow, 0)
    for k in range(PER_T // ZR):
        pltpu.sync_copy(zbuf, s_sh.at[pl.ds(sid * PER_T + k * ZR, ZR)])
    pltpu.sync_copy(srcp_hbm.at[pl.ds(wid * NB, NB)], sidx)
    pltpu.sync_copy(dstp_hbm.at[pl.ds(wid * NB, NB)], didx)
    plsc.subcore_barrier()

    pltpu.async_copy(g_hbm.at[sidx.at[0]], bufa, sem_a)

    def body(j, c):
        b0 = 2 * j
        b1 = 2 * j + 1
        pltpu.make_async_copy(g_hbm.at[sidx.at[b0]], bufa, sem_a).wait()
        pltpu.async_copy(g_hbm.at[sidx.at[b1]], bufb, sem_b)
        pltpu.sync_copy(bufa, s_sh.at[didx.at[b0]], add=True)
        pltpu.make_async_copy(g_hbm.at[sidx.at[b1]], bufb, sem_b).wait()
        nxt = jnp.minimum(b1 + 1, NB - 1)
        pltpu.async_copy(g_hbm.at[sidx.at[nxt]], bufa, sem_a)
        pltpu.sync_copy(bufb, s_sh.at[didx.at[b1]], add=True)
        return c

    lax.fori_loop(0, NB // 2, body, 0)
    pltpu.make_async_copy(g_hbm.at[sidx.at[NB - 1]], bufa, sem_a).wait()
    plsc.subcore_barrier()
    pltpu.sync_copy(s_sh.at[pl.ds(sid * PER_T, PER_T)],
                    out_hbm.at[cid, pl.ds(sid * PER_T, PER_T)])


def _g_body(x_ref, w_ref, p0_ref, p1_ref, o_ref):
    dinv = lax.rsqrt(p0_ref[...] + p1_ref[...] + 1.0)
    h = lax.dot_general(x_ref[...], w_ref[...], (((1,), (1,)), ((), ())),
                        preferred_element_type=jnp.float32)
    o_ref[...] = h * dinv


def _fin_body(x_ref, s0_ref, s1_ref, g_ref, p0_ref, p1_ref, b_ref, a_ref,
              o_ref):
    dinv = lax.rsqrt(p0_ref[...] + p1_ref[...] + 1.0)
    a = a_ref[0, 0]
    aggr = dinv * (s0_ref[0] + s1_ref[0] + g_ref[...]) + b_ref[...]
    o_ref[...] = a * x_ref[...] + (1.0 - a) * aggr


def kernel(node_features, edge_index, W, b, alpha):
    x = node_features
    n, d = x.shape
    e = edge_index.shape[1]
    src = edge_index[0]
    dst = edge_index[1]

    pad = EP - e
    pidx = jnp.arange(pad, dtype=jnp.int32)
    src_p = jnp.concatenate([src, pidx % n]).reshape(NW * NB, LANE)
    dst_p = jnp.concatenate([dst, n + (pidx % 16)]).reshape(NW * NB, LANE)

    mesh = plsc.VectorSubcoreMesh(core_axis_name="c", subcore_axis_name="s")

    deg_parts = pl.kernel(
        _deg_body,
        out_type=jax.ShapeDtypeStruct((NSC, NP), jnp.float32),
        mesh=mesh,
        scratch_types=[
            pltpu.VMEM((NB, LANE), jnp.int32),
            pltpu.VMEM((LANE,), jnp.float32),
            pltpu.VMEM((PER_T,), jnp.float32),
            pltpu.VMEM_SHARED((NP,), jnp.float32),
        ],
    )(dst_p)

    p0 = deg_parts[0, :n, None]
    p1 = deg_parts[1, :n, None]

    g = pl.pallas_call(
        _g_body,
        grid=(n // BR,),
        in_specs=[
            pl.BlockSpec((BR, d), lambda i: (i, 0)),
            pl.BlockSpec((d, d), lambda i: (0, 0)),
            pl.BlockSpec((BR, 1), lambda i: (i, 0)),
            pl.BlockSpec((BR, 1), lambda i: (i, 0)),
        ],
        out_specs=pl.BlockSpec((BR, d), lambda i: (i, 0)),
        out_shape=jax.ShapeDtypeStruct((n, d), jnp.float32),
    )(x, W, p0, p1)

    s_parts = pl.kernel(
        _scat_body,
        out_type=jax.ShapeDtypeStruct((NSC, NP, d), jnp.float32),
        mesh=mesh,
        scratch_types=[
            pltpu.VMEM((NB, LANE), jnp.int32),
            pltpu.VMEM((NB, LANE), jnp.int32),
            pltpu.VMEM((LANE, d), jnp.float32),
            pltpu.VMEM((LANE, d), jnp.float32),
            pltpu.VMEM((ZR, d), jnp.float32),
            pltpu.VMEM_SHARED((NP, d), jnp.float32),
            pltpu.SemaphoreType.DMA,
            pltpu.SemaphoreType.DMA,
        ],
    )(g, src_p, dst_p)

    out = pl.pallas_call(
        _fin_body,
        grid=(n // BR,),
        in_specs=[
            pl.BlockSpec((BR, d), lambda i: (i, 0)),
            pl.BlockSpec((1, BR, d), lambda i: (0, i, 0)),
            pl.BlockSpec((1, BR, d), lambda i: (1, i, 0)),
            pl.BlockSpec((BR, d), lambda i: (i, 0)),
            pl.BlockSpec((BR, 1), lambda i: (i, 0)),
            pl.BlockSpec((BR, 1), lambda i: (i, 0)),
            pl.BlockSpec((1, d), lambda i: (0, 0)),
            pl.BlockSpec((1, 1), lambda i: (0, 0)),
        ],
        out_specs=pl.BlockSpec((BR, d), lambda i: (i, 0)),
        out_shape=jax.ShapeDtypeStruct((n, d), jnp.float32),
    )(x, s_parts, s_parts, g, p0, p1, b.reshape(1, d), alpha.reshape(1, 1))
    return out


# speedup vs baseline: 34.0062x; 34.0062x over previous
import functools

import jax
import jax.numpy as jnp
from jax import lax
from jax.experimental import pallas as pl
from jax.experimental.pallas import tpu as pltpu
from jax.experimental.pallas import tpu_sc as plsc

LANE = 128
NSC = 2
NTILE = 16
NW = NSC * NTILE
NB = 80
EP = NW * NB * LANE
NP = 10240
PER_T = NP // NTILE
ZR = 128
BR = 1000


def _deg_body(dstp_hbm, outdeg_hbm, didx, ones_v, zvec, deg_sh):
    cid = lax.axis_index("c")
    sid = lax.axis_index("s")
    wid = cid * NTILE + sid
    zero = jnp.zeros((16,), jnp.float32)
    one = jnp.ones((16,), jnp.float32)
    for i in range(PER_T // 16):
        zvec[pl.ds(i * 16, 16)] = zero
    for i in range(LANE // 16):
        ones_v[pl.ds(i * 16, 16)] = one
    pltpu.sync_copy(zvec, deg_sh.at[pl.ds(sid * PER_T, PER_T)])
    pltpu.sync_copy(dstp_hbm.at[pl.ds(wid * NB, NB)], didx)
    plsc.subcore_barrier()

    def body(j, c):
        pltpu.sync_copy(ones_v, deg_sh.at[didx.at[j]], add=True)
        return c

    lax.fori_loop(0, NB, body, 0)
    plsc.subcore_barrier()

    @pl.when(sid == 0)
    def _():
        pltpu.sync_copy(deg_sh, outdeg_hbm.at[cid])


def _scat_body(g_hbm, srcp_hbm, dstp_hbm, out_hbm,
               sidx, didx, bufa, bufb, s_sh, sem_a, sem_b):
    cid = lax.axis_index("c")
    sid = lax.axis_index("s")
    wid = cid * NTILE + sid
    zero = jnp.zeros((16,), jnp.float32)

    def zrow(i, c):
        for k in range(LANE // 16):
            bufa[i, pl.ds(k * 16, 16)] = zero
        return c

    lax.fori_loop(0, ZR, zrow, 0)
    for k in range(PER_T // ZR):
        pltpu.sync_copy(bufa, s_sh.at[pl.ds(sid * PER_T + k * ZR, ZR)])
    pltpu.sync_copy(srcp_hbm.at[pl.ds(wid * NB, NB)], sidx)
    pltpu.sync_copy(dstp_hbm.at[pl.ds(wid * NB, 4)], didx)
    pltpu.async_copy(g_hbm.at[sidx.at[0]], bufa, sem_a)
    plsc.subcore_barrier()

    def body(j, c):
        b0 = 2 * j
        b1 = 2 * j + 1
        p2 = 2 * lax.rem(j, 2)
        pltpu.make_async_copy(g_hbm.at[sidx.at[b0]], bufa, sem_a).wait()
        pltpu.async_copy(g_hbm.at[sidx.at[b1]], bufb, sem_b)
        pltpu.sync_copy(bufa, s_sh.at[didx.at[p2]], add=True)
        pltpu.make_async_copy(g_hbm.at[sidx.at[b1]], bufb, sem_b).wait()
        nxt = jnp.minimum(b1 + 1, NB - 1)
        pltpu.async_copy(g_hbm.at[sidx.at[nxt]], bufa, sem_a)
        pltpu.sync_copy(bufb, s_sh.at[didx.at[p2 + 1]], add=True)
        pre = jnp.minimum(b0 + 4, NB - 2)
        pltpu.sync_copy(dstp_hbm.at[pl.ds(wid * NB + pre, 2)],
                        didx.at[pl.ds(p2, 2)])
        return c

    lax.fori_loop(0, NB // 2, body, 0)
    pltpu.make_async_copy(g_hbm.at[sidx.at[NB - 1]], bufa, sem_a).wait()
    plsc.subcore_barrier()
    pltpu.sync_copy(s_sh.at[pl.ds(sid * PER_T, PER_T)],
                    out_hbm.at[cid, pl.ds(sid * PER_T, PER_T)])


def _g_body(x_ref, w_ref, p0_ref, p1_ref, o_ref):
    dinv = lax.rsqrt(p0_ref[...] + p1_ref[...] + 1.0)
    h = lax.dot_general(x_ref[...], w_ref[...], (((1,), (1,)), ((), ())),
                        preferred_element_type=jnp.float32)
    o_ref[...] = h * dinv


def _fin_body(x_ref, s0_ref, s1_ref, g_ref, p0_ref, p1_ref, b_ref, a_ref,
              o_ref):
    dinv = lax.rsqrt(p0_ref[...] + p1_ref[...] + 1.0)
    a = a_ref[0, 0]
    aggr = dinv * (s0_ref[0] + s1_ref[0] + g_ref[...]) + b_ref[...]
    o_ref[...] = a * x_ref[...] + (1.0 - a) * aggr


def kernel(node_features, edge_index, W, b, alpha):
    x = node_features
    n, d = x.shape
    e = edge_index.shape[1]
    src = edge_index[0]
    dst = edge_index[1]

    pad = EP - e
    pidx = jnp.arange(pad, dtype=jnp.int32)
    src_p = jnp.concatenate([src, pidx % n]).reshape(NW * NB, LANE)
    dst_p = jnp.concatenate([dst, n + (pidx % 16)]).reshape(NW * NB, LANE)

    mesh = plsc.VectorSubcoreMesh(core_axis_name="c", subcore_axis_name="s")

    deg_parts = pl.kernel(
        _deg_body,
        out_type=jax.ShapeDtypeStruct((NSC, NP), jnp.float32),
        mesh=mesh,
        scratch_types=[
            pltpu.VMEM((NB, LANE), jnp.int32),
            pltpu.VMEM((LANE,), jnp.float32),
            pltpu.VMEM((PER_T,), jnp.float32),
            pltpu.VMEM_SHARED((NP,), jnp.float32),
        ],
    )(dst_p)

    p0 = deg_parts[0, :n, None]
    p1 = deg_parts[1, :n, None]

    g = pl.pallas_call(
        _g_body,
        grid=(n // BR,),
        in_specs=[
            pl.BlockSpec((BR, d), lambda i: (i, 0)),
            pl.BlockSpec((d, d), lambda i: (0, 0)),
            pl.BlockSpec((BR, 1), lambda i: (i, 0)),
            pl.BlockSpec((BR, 1), lambda i: (i, 0)),
        ],
        out_specs=pl.BlockSpec((BR, d), lambda i: (i, 0)),
        out_shape=jax.ShapeDtypeStruct((n, d), jnp.float32),
    )(x, W, p0, p1)

    s_parts = pl.kernel(
        _scat_body,
        out_type=jax.ShapeDtypeStruct((NSC, NP, d), jnp.float32),
        mesh=mesh,
        scratch_types=[
            pltpu.VMEM((NB, LANE), jnp.int32),
            pltpu.VMEM((4, LANE), jnp.int32),
            pltpu.VMEM((LANE, d), jnp.float32),
            pltpu.VMEM((LANE, d), jnp.float32),
            pltpu.VMEM_SHARED((NP, d), jnp.float32),
            pltpu.SemaphoreType.DMA,
            pltpu.SemaphoreType.DMA,
        ],
    )(g, src_p, dst_p)

    out = pl.pallas_call(
        _fin_body,
        grid=(n // BR,),
        in_specs=[
            pl.BlockSpec((BR, d), lambda i: (i, 0)),
            pl.BlockSpec((1, BR, d), lambda i: (0, i, 0)),
            pl.BlockSpec((1, BR, d), lambda i: (1, i, 0)),
            pl.BlockSpec((BR, d), lambda i: (i, 0)),
            pl.BlockSpec((BR, 1), lambda i: (i, 0)),
            pl.BlockSpec((BR, 1), lambda i: (i, 0)),
            pl.BlockSpec((1, d), lambda i: (0, 0)),
            pl.BlockSpec((1, 1), lambda i: (0, 0)),
        ],
        out_specs=pl.BlockSpec((BR, d), lambda i: (i, 0)),
        out_shape=jax.ShapeDtypeStruct((n, d), jnp.float32),
    )(x, s_parts, s_parts, g, p0, p1, b.reshape(1, d), alpha.reshape(1, 1))
    return out


# speedup vs baseline: 34.6816x; 1.0199x over previous
import functools

import jax
import jax.numpy as jnp
from jax import lax
from jax.experimental import pallas as pl
from jax.experimental.pallas import tpu as pltpu
from jax.experimental.pallas import tpu_sc as plsc

LANE = 128
NSC = 2
NTILE = 16
NW = NSC * NTILE
NB = 80
EP = NW * NB * LANE
NP = 10240
PER_T = NP // NTILE
ZR = 128
BR = 1000


def _deg_body(dstp_hbm, outdeg_hbm, didx, ones_v, zvec, deg_sh, sem_d):
    cid = lax.axis_index("c")
    sid = lax.axis_index("s")
    wid = cid * NTILE + sid
    zero = jnp.zeros((16,), jnp.float32)
    one = jnp.ones((16,), jnp.float32)
    for i in range(PER_T // 16):
        zvec[pl.ds(i * 16, 16)] = zero
    for i in range(LANE // 16):
        ones_v[pl.ds(i * 16, 16)] = one
    pltpu.sync_copy(zvec, deg_sh.at[pl.ds(sid * PER_T, PER_T)])
    pltpu.sync_copy(dstp_hbm.at[pl.ds(wid * NB, NB)], didx)
    plsc.subcore_barrier()

    def issue(j, c):
        pltpu.async_copy(ones_v, deg_sh.at[didx.at[j]], sem_d, add=True)
        return c

    lax.fori_loop(0, NB, issue, 0)

    def drain(j, c):
        pltpu.make_async_copy(ones_v, deg_sh.at[didx.at[0]], sem_d).wait()
        return c

    lax.fori_loop(0, NB, drain, 0)
    plsc.subcore_barrier()

    @pl.when(sid == 0)
    def _():
        pltpu.sync_copy(deg_sh, outdeg_hbm.at[cid])


def _scat_body(g_hbm, srcp_hbm, dstp_hbm, out_hbm,
               sidx, dsa, dsb, bufa, bufb, s_sh, sem_a, sem_b, sem_da, sem_db):
    cid = lax.axis_index("c")
    sid = lax.axis_index("s")
    wid = cid * NTILE + sid
    base = wid * NB
    zero = jnp.zeros((16,), jnp.float32)

    def zrow(i, c):
        for k in range(LANE // 16):
            bufa[i, pl.ds(k * 16, 16)] = zero
        return c

    lax.fori_loop(0, ZR, zrow, 0)
    for k in range(PER_T // ZR):
        pltpu.sync_copy(bufa, s_sh.at[pl.ds(sid * PER_T + k * ZR, ZR)])

    pltpu.sync_copy(srcp_hbm.at[pl.ds(base, NB)], sidx)
    pltpu.async_copy(dstp_hbm.at[pl.ds(base, 2)], dsa, sem_da)
    pltpu.async_copy(dstp_hbm.at[pl.ds(base + 2, 2)], dsb, sem_db)
    pltpu.async_copy(g_hbm.at[sidx.at[0]], bufa, sem_a)
    plsc.subcore_barrier()

    def pair(q0, buf_first, buf_second, dslot, sem_first, sem_second):
        q1 = q0 + 1
        pltpu.make_async_copy(g_hbm.at[sidx.at[q0]], buf_first,
                              sem_first).wait()
        pltpu.async_copy(g_hbm.at[sidx.at[q1]], buf_second, sem_second)
        pltpu.sync_copy(buf_first, s_sh.at[dslot.at[0]], add=True)
        pltpu.make_async_copy(g_hbm.at[sidx.at[q1]], buf_second,
                              sem_second).wait()
        nxt = jnp.minimum(q1 + 1, NB - 1)
        pltpu.async_copy(g_hbm.at[sidx.at[nxt]], buf_first, sem_first)
        pltpu.sync_copy(buf_second, s_sh.at[dslot.at[1]], add=True)

    def body(j, c):
        q = 4 * j
        pltpu.make_async_copy(dstp_hbm.at[pl.ds(base, 2)], dsa,
                              sem_da).wait()
        pair(q, bufa, bufb, dsa, sem_a, sem_b)
        pre_a = jnp.minimum(q + 4, NB - 2)
        pltpu.async_copy(dstp_hbm.at[pl.ds(base + pre_a, 2)], dsa, sem_da)
        pltpu.make_async_copy(dstp_hbm.at[pl.ds(base, 2)], dsb,
                              sem_db).wait()
        pair(q + 2, bufa, bufb, dsb, sem_a, sem_b)
        pre_b = jnp.minimum(q + 6, NB - 2)
        pltpu.async_copy(dstp_hbm.at[pl.ds(base + pre_b, 2)], dsb, sem_db)
        return c

    lax.fori_loop(0, NB // 4, body, 0)
    pltpu.make_async_copy(g_hbm.at[sidx.at[NB - 1]], bufa, sem_a).wait()
    pltpu.make_async_copy(dstp_hbm.at[pl.ds(base, 2)], dsa, sem_da).wait()
    pltpu.make_async_copy(dstp_hbm.at[pl.ds(base, 2)], dsb, sem_db).wait()
    plsc.subcore_barrier()
    pltpu.sync_copy(s_sh.at[pl.ds(sid * PER_T, PER_T)],
                    out_hbm.at[cid, pl.ds(sid * PER_T, PER_T)])


def _g_body(x_ref, w_ref, p0_ref, p1_ref, o_ref):
    dinv = lax.rsqrt(p0_ref[...] + p1_ref[...] + 1.0)
    h = lax.dot_general(x_ref[...], w_ref[...], (((1,), (1,)), ((), ())),
                        preferred_element_type=jnp.float32)
    o_ref[...] = h * dinv


def _fin_body(x_ref, s0_ref, s1_ref, g_ref, p0_ref, p1_ref, b_ref, a_ref,
              o_ref):
    dinv = lax.rsqrt(p0_ref[...] + p1_ref[...] + 1.0)
    a = a_ref[0, 0]
    aggr = dinv * (s0_ref[0] + s1_ref[0] + g_ref[...]) + b_ref[...]
    o_ref[...] = a * x_ref[...] + (1.0 - a) * aggr


def kernel(node_features, edge_index, W, b, alpha):
    x = node_features
    n, d = x.shape
    e = edge_index.shape[1]
    src = edge_index[0]
    dst = edge_index[1]

    pad = EP - e
    pidx = jnp.arange(pad, dtype=jnp.int32)
    src_p = jnp.concatenate([src, pidx % n]).reshape(NW * NB, LANE)
    dst_p = jnp.concatenate([dst, n + (pidx % 16)]).reshape(NW * NB, LANE)

    mesh = plsc.VectorSubcoreMesh(core_axis_name="c", subcore_axis_name="s")

    deg_parts = pl.kernel(
        _deg_body,
        out_type=jax.ShapeDtypeStruct((NSC, NP), jnp.float32),
        mesh=mesh,
        scratch_types=[
            pltpu.VMEM((NB, LANE), jnp.int32),
            pltpu.VMEM((LANE,), jnp.float32),
            pltpu.VMEM((PER_T,), jnp.float32),
            pltpu.VMEM_SHARED((NP,), jnp.float32),
            pltpu.SemaphoreType.DMA,
        ],
    )(dst_p)

    p0 = deg_parts[0, :n, None]
    p1 = deg_parts[1, :n, None]

    g = pl.pallas_call(
        _g_body,
        grid=(n // BR,),
        in_specs=[
            pl.BlockSpec((BR, d), lambda i: (i, 0)),
            pl.BlockSpec((d, d), lambda i: (0, 0)),
            pl.BlockSpec((BR, 1), lambda i: (i, 0)),
            pl.BlockSpec((BR, 1), lambda i: (i, 0)),
        ],
        out_specs=pl.BlockSpec((BR, d), lambda i: (i, 0)),
        out_shape=jax.ShapeDtypeStruct((n, d), jnp.float32),
    )(x, W, p0, p1)

    s_parts = pl.kernel(
        _scat_body,
        out_type=jax.ShapeDtypeStruct((NSC, NP, d), jnp.float32),
        mesh=mesh,
        scratch_types=[
            pltpu.VMEM((NB, LANE), jnp.int32),
            pltpu.VMEM((2, LANE), jnp.int32),
            pltpu.VMEM((2, LANE), jnp.int32),
            pltpu.VMEM((LANE, d), jnp.float32),
            pltpu.VMEM((LANE, d), jnp.float32),
            pltpu.VMEM_SHARED((NP, d), jnp.float32),
            pltpu.SemaphoreType.DMA,
            pltpu.SemaphoreType.DMA,
            pltpu.SemaphoreType.DMA,
            pltpu.SemaphoreType.DMA,
        ],
    )(g, src_p, dst_p)

    out = pl.pallas_call(
        _fin_body,
        grid=(n // BR,),
        in_specs=[
            pl.BlockSpec((BR, d), lambda i: (i, 0)),
            pl.BlockSpec((1, BR, d), lambda i: (0, i, 0)),
            pl.BlockSpec((1, BR, d), lambda i: (1, i, 0)),
            pl.BlockSpec((BR, d), lambda i: (i, 0)),
            pl.BlockSpec((BR, 1), lambda i: (i, 0)),
            pl.BlockSpec((BR, 1), lambda i: (i, 0)),
            pl.BlockSpec((1, d), lambda i: (0, 0)),
            pl.BlockSpec((1, 1), lambda i: (0, 0)),
        ],
        out_specs=pl.BlockSpec((BR, d), lambda i: (i, 0)),
        out_shape=jax.ShapeDtypeStruct((n, d), jnp.float32),
    )(x, s_parts, s_parts, g, p0, p1, b.reshape(1, d), alpha.reshape(1, 1))
    return out
